# Initial kernel scaffold; baseline (speedup 1.0000x reference)
import jax
import jax.numpy as jnp
from jax.experimental import pallas as pl


def kernel(concept_embedding, relation_embedding, edge_index, edge_relation, edge_weight, W_msg, W_self, W_upd):
    raise NotImplementedError("write your pallas kernel here")


# speedup vs baseline: 2.2900x; 2.2900x over previous
import functools

import jax
import jax.numpy as jnp
from jax import lax
from jax.experimental import pallas as pl
from jax.experimental.pallas import tpu as pltpu
from jax.experimental.pallas import tpu_sc as plsc

N = 10000
E = 160000
D = 128
R = 40

NC, NS, L = 2, 16, 16
NW = NC * NS
EPW = E // NW
C = 128
NCHUNK = (EPW + C - 1) // C
LAST_BASE = EPW - C
OVERLAP = NCHUNK * C - EPW
G = D // L
NPAD = N + L
ZB = 632


def _mm_body(a_ref, w_ref, o_ref):
    o_ref[...] = jnp.dot(a_ref[...], w_ref[...],
                         preferred_element_type=jnp.float32)


def _tc_matmul(a, w, bm):
    m = a.shape[0]
    return pl.pallas_call(
        _mm_body,
        grid=(m // bm,),
        in_specs=[pl.BlockSpec((bm, D), lambda i: (i, 0)),
                  pl.BlockSpec((D, D), lambda i: (0, 0))],
        out_specs=pl.BlockSpec((bm, D), lambda i: (i, 0)),
        out_shape=jax.ShapeDtypeStruct((m, D), jnp.float32),
    )(a, w)


def _upd_body(agg_ref, x_ref, wu_ref, ws_ref, o_ref):
    a = agg_ref[0] + agg_ref[1]
    o_ref[...] = jnp.maximum(
        jnp.dot(a, wu_ref[...], preferred_element_type=jnp.float32)
        + jnp.dot(x_ref[...], ws_ref[...], preferred_element_type=jnp.float32),
        0.0)


def _tc_update(aggp, x, w_upd, w_self, bm=400):
    return pl.pallas_call(
        _upd_body,
        grid=(N // bm,),
        in_specs=[pl.BlockSpec((NC, bm, D), lambda i: (0, i, 0)),
                  pl.BlockSpec((bm, D), lambda i: (i, 0)),
                  pl.BlockSpec((D, D), lambda i: (0, 0)),
                  pl.BlockSpec((D, D), lambda i: (0, 0))],
        out_specs=pl.BlockSpec((bm, D), lambda i: (i, 0)),
        out_shape=jax.ShapeDtypeStruct((N, D), jnp.float32),
    )(aggp, x, w_upd, w_self)


def _sc_agg(xrm, src, dst, rel, w):
    mesh = plsc.VectorSubcoreMesh(core_axis_name="c", subcore_axis_name="s")

    @functools.partial(
        pl.kernel,
        out_type=jax.ShapeDtypeStruct((NC, N, D), jnp.float32),
        mesh=mesh,
        scratch_types=[
            pltpu.VMEM((C, D), jnp.float32),
            pltpu.VMEM((C, D), jnp.float32),
            pltpu.VMEM((C,), jnp.int32),
            pltpu.VMEM((C,), jnp.int32),
            pltpu.VMEM((C,), jnp.int32),
            pltpu.VMEM((C,), jnp.float32),
            pltpu.VMEM_SHARED((NPAD, D), jnp.float32),
            pltpu.SemaphoreType.DMA,
            pltpu.SemaphoreType.DMA,
        ],
    )
    def k(xrm_hbm, src_hbm, dst_hbm, rel_hbm, w_hbm, out_hbm,
          buf, bufe, sidx, didx, relv, wv, agg_sh, sem, sem2):
        cid = lax.axis_index("c")
        sid = lax.axis_index("s")
        wid = sid * NC + cid

        def zrow(i, _):
            r = i // G
            g = i % G
            buf[r, pl.ds(g * L, L)] = jnp.zeros((L,), jnp.float32)
            return 0
        lax.fori_loop(0, C * G, zrow, 0)
        base_z = sid * ZB
        for b in range(ZB // C):
            pltpu.sync_copy(buf, agg_sh.at[pl.ds(base_z + b * C, C)])

        @pl.when(sid < NS - 1)
        def _ztail_full():
            pltpu.sync_copy(buf.at[pl.ds(0, ZB % C)],
                            agg_sh.at[pl.ds(base_z + (ZB // C) * C, ZB % C)])

        @pl.when(sid == NS - 1)
        def _ztail_last():
            rem = NPAD - (NS - 1) * ZB - (ZB // C) * C
            pltpu.sync_copy(buf.at[pl.ds(0, rem)],
                            agg_sh.at[pl.ds(base_z + (ZB // C) * C, rem)])

        plsc.subcore_barrier()

        ebase = wid * EPW

        def chunk(kk, _):
            gb = ebase + lax.min(kk * C, LAST_BASE)
            pltpu.sync_copy(src_hbm.at[pl.ds(gb, C)], sidx)
            pltpu.sync_copy(dst_hbm.at[pl.ds(gb, C)], didx)
            pltpu.sync_copy(rel_hbm.at[pl.ds(gb, C)], relv)
            pltpu.sync_copy(w_hbm.at[pl.ds(gb, C)], wv)

            for j in range(C // L):
                relv[pl.ds(j * L, L)] = relv[pl.ds(j * L, L)] + N

            @pl.when(kk == NCHUNK - 1)
            def _fix():
                for j in range(C // L):
                    v = didx[pl.ds(j * L, L)]
                    lanes = j * L + lax.iota(jnp.int32, L)
                    didx[pl.ds(j * L, L)] = jnp.where(
                        lanes >= OVERLAP, v, jnp.full((L,), N, jnp.int32))

            cx = pltpu.async_copy(xrm_hbm.at[sidx], buf, sem)
            ce = pltpu.async_copy(xrm_hbm.at[relv], bufe, sem2)
            cx.wait()
            ce.wait()

            def sub(jb, _):
                wv16 = wv[pl.ds(jb * L, L)]
                for i in range(L):
                    w_b = jnp.full((L,), wv16[i], jnp.float32)
                    e = jb * L + i
                    for g in range(G):
                        xv = buf[e, pl.ds(g * L, L)]
                        ev = bufe[e, pl.ds(g * L, L)]
                        buf[e, pl.ds(g * L, L)] = jnp.maximum(
                            xv + w_b * ev, 0.0)
                return 0
            lax.fori_loop(0, C // L, sub, 0)

            pltpu.sync_copy(buf, agg_sh.at[didx], add=True)
            return 0

        lax.fori_loop(0, NCHUNK, chunk, 0)
        plsc.subcore_barrier()

        base_o = sid * ZB

        @pl.when(sid < NS - 1)
        def _w_full():
            pltpu.sync_copy(agg_sh.at[pl.ds(base_o, ZB)],
                            out_hbm.at[cid, pl.ds(base_o, ZB)])

        @pl.when(sid == NS - 1)
        def _w_last():
            rows = N - (NS - 1) * ZB
            pltpu.sync_copy(agg_sh.at[pl.ds(base_o, rows)],
                            out_hbm.at[cid, pl.ds(base_o, rows)])

    return k(xrm, src, dst, rel, w)


def _sc_triples(xout, rel_emb, src, dst, rel, w):
    mesh = plsc.VectorSubcoreMesh(core_axis_name="c", subcore_axis_name="s")

    @functools.partial(
        pl.kernel,
        out_type=jax.ShapeDtypeStruct((3 * E, D), jnp.float32),
        mesh=mesh,
        scratch_types=[
            pltpu.VMEM((C, D), jnp.float32),
            pltpu.VMEM((C, D), jnp.float32),
            pltpu.VMEM((C, D), jnp.float32),
            pltpu.VMEM((C,), jnp.int32),
            pltpu.VMEM((C,), jnp.int32),
            pltpu.VMEM((C,), jnp.int32),
            pltpu.VMEM((C,), jnp.float32),
            pltpu.VMEM((C,), jnp.int32),
            pltpu.VMEM((C,), jnp.int32),
            pltpu.VMEM((C,), jnp.int32),
            pltpu.SemaphoreType.DMA,
            pltpu.SemaphoreType.DMA,
            pltpu.SemaphoreType.DMA,
        ],
    )
    def k(xout_hbm, emr_hbm, src_hbm, dst_hbm, rel_hbm, w_hbm, out_hbm,
          bufs, bufd, bufa, sidx, didx, relv, wv,
          oxs, oxa, oxd, sem1, sem2, sem3):
        cid = lax.axis_index("c")
        sid = lax.axis_index("s")
        wid = sid * NC + cid
        ebase = wid * EPW

        def chunk(kk, _):
            gb = ebase + lax.min(kk * C, LAST_BASE)
            pltpu.sync_copy(src_hbm.at[pl.ds(gb, C)], sidx)
            pltpu.sync_copy(dst_hbm.at[pl.ds(gb, C)], didx)
            pltpu.sync_copy(rel_hbm.at[pl.ds(gb, C)], relv)
            pltpu.sync_copy(w_hbm.at[pl.ds(gb, C)], wv)
            cs = pltpu.async_copy(xout_hbm.at[sidx], bufs, sem1)
            cd = pltpu.async_copy(xout_hbm.at[didx], bufd, sem2)
            ca = pltpu.async_copy(emr_hbm.at[relv], bufa, sem3)

            for j in range(C // L):
                rows3 = (gb + j * L + lax.iota(jnp.int32, L)) * 3
                oxs[pl.ds(j * L, L)] = rows3
                oxa[pl.ds(j * L, L)] = rows3 + 1
                oxd[pl.ds(j * L, L)] = rows3 + 2

            ca.wait()

            def sub(jb, _):
                wv16 = wv[pl.ds(jb * L, L)]
                for i in range(L):
                    w_b = jnp.full((L,), wv16[i], jnp.float32)
                    e = jb * L + i
                    for g in range(G):
                        bufa[e, pl.ds(g * L, L)] = (
                            w_b * bufa[e, pl.ds(g * L, L)])
                return 0
            lax.fori_loop(0, C // L, sub, 0)
            cs.wait()
            cd.wait()
            pltpu.sync_copy(bufs, out_hbm.at[oxs])
            pltpu.sync_copy(bufa, out_hbm.at[oxa])
            pltpu.sync_copy(bufd, out_hbm.at[oxd])
            return 0

        lax.fori_loop(0, NCHUNK, chunk, 0)

    return k(xout, rel_emb, src, dst, rel, w)


def kernel(concept_embedding, relation_embedding, edge_index, edge_relation,
           edge_weight, W_msg, W_self, W_upd):
    x = concept_embedding
    src = edge_index[0]
    dst = edge_index[1]
    xr = jnp.concatenate(
        [x, relation_embedding,
         jnp.zeros((10240 - N - R, D), jnp.float32)], axis=0)
    xrm = _tc_matmul(xr, W_msg, bm=512)
    aggp = _sc_agg(xrm, src, dst, edge_relation, edge_weight)
    xout = _tc_update(aggp, x, W_upd, W_self)
    trip = _sc_triples(xout, relation_embedding, src, dst,
                       edge_relation, edge_weight)
    return trip.reshape(E, 3 * D)


# speedup vs baseline: 2.5334x; 1.1063x over previous
import functools

import jax
import jax.numpy as jnp
from jax import lax
from jax.experimental import pallas as pl
from jax.experimental.pallas import tpu as pltpu
from jax.experimental.pallas import tpu_sc as plsc

N = 10000
E = 160000
D = 128
R = 40

NC, NS, L = 2, 16, 16
NW = NC * NS
EPW = E // NW
C = 128
NCHUNK = (EPW + C - 1) // C
LAST_BASE = EPW - C
OVERLAP = NCHUNK * C - EPW
G = D // L
ZB = 632
NPAD = NS * ZB
CA = 96
NCHA = 54


def _mm_body(a_ref, w_ref, o_ref):
    o_ref[...] = jnp.dot(a_ref[...], w_ref[...],
                         preferred_element_type=jnp.float32)


def _tc_matmul(a, w, bm):
    m = a.shape[0]
    return pl.pallas_call(
        _mm_body,
        grid=(m // bm,),
        in_specs=[pl.BlockSpec((bm, D), lambda i: (i, 0)),
                  pl.BlockSpec((D, D), lambda i: (0, 0))],
        out_specs=pl.BlockSpec((bm, D), lambda i: (i, 0)),
        out_shape=jax.ShapeDtypeStruct((m, D), jnp.float32),
    )(a, w)


def _upd_body(agg_ref, x_ref, wu_ref, ws_ref, o_ref):
    a = agg_ref[0] + agg_ref[1]
    o_ref[...] = jnp.maximum(
        jnp.dot(a, wu_ref[...], preferred_element_type=jnp.float32)
        + jnp.dot(x_ref[...], ws_ref[...], preferred_element_type=jnp.float32),
        0.0)


def _tc_update(aggp, x, w_upd, w_self, bm=400):
    return pl.pallas_call(
        _upd_body,
        grid=(N // bm,),
        in_specs=[pl.BlockSpec((NC, bm, D), lambda i: (0, i, 0)),
                  pl.BlockSpec((bm, D), lambda i: (i, 0)),
                  pl.BlockSpec((D, D), lambda i: (0, 0)),
                  pl.BlockSpec((D, D), lambda i: (0, 0))],
        out_specs=pl.BlockSpec((bm, D), lambda i: (i, 0)),
        out_shape=jax.ShapeDtypeStruct((N, D), jnp.float32),
    )(aggp, x, w_upd, w_self)


def _sc_agg(xrm, src, dst, rel, w):
    mesh = plsc.VectorSubcoreMesh(core_axis_name="c", subcore_axis_name="s")

    @functools.partial(
        pl.kernel,
        out_type=jax.ShapeDtypeStruct((NC, N, D), jnp.float32),
        mesh=mesh,
        scratch_types=[
            pltpu.VMEM((R, D), jnp.float32),
            pltpu.VMEM((EPW,), jnp.int32),
            pltpu.VMEM((EPW,), jnp.int32),
            pltpu.VMEM((EPW,), jnp.float32),
            [pltpu.VMEM((CA, D), jnp.float32) for _ in range(2)],
            [pltpu.VMEM((CA,), jnp.int32) for _ in range(2)],
            [pltpu.VMEM((CA,), jnp.int32) for _ in range(2)],
            [pltpu.SemaphoreType.DMA for _ in range(2)],
            [pltpu.SemaphoreType.DMA for _ in range(2)],
            pltpu.VMEM_SHARED((NPAD, D), jnp.float32),
        ],
    )
    def k(xrm_hbm, src_hbm, dst_hbm, rel_hbm, w_hbm, out_hbm,
          em_v, pdst, prel, pw, bufs, gidx, sdix, gsem, ssem, agg_sh):
        buf = bufs[0]
        cid = lax.axis_index("c")
        sid = lax.axis_index("s")
        wid = sid * NC + cid

        def zrow(i, _):
            r = i // G
            g = i % G
            buf[r, pl.ds(g * L, L)] = jnp.zeros((L,), jnp.float32)
            return 0
        lax.fori_loop(0, CA * G, zrow, 0)
        base_z = sid * ZB
        for b in range(ZB // CA):
            pltpu.sync_copy(buf, agg_sh.at[pl.ds(base_z + b * CA, CA)])
        pltpu.sync_copy(buf.at[pl.ds(0, ZB % CA)],
                        agg_sh.at[pl.ds(base_z + (ZB // CA) * CA, ZB % CA)])
        plsc.subcore_barrier()

        eb = wid * EPW
        pltpu.sync_copy(dst_hbm.at[pl.ds(eb, EPW)], pdst)
        pltpu.sync_copy(rel_hbm.at[pl.ds(eb, EPW)], prel)
        pltpu.sync_copy(w_hbm.at[pl.ds(eb, EPW)], pw)
        pltpu.sync_copy(xrm_hbm.at[pl.ds(N, R)], em_v)

        pltpu.sync_copy(src_hbm.at[pl.ds(eb, CA)], gidx[0])
        pltpu.async_copy(xrm_hbm.at[gidx[0]], bufs[0], gsem[0])

        def body(k2, _):
            for b in range(2):
                c = k2 * 2 + b
                cb = lax.min(c * CA, EPW - CA)
                ob = 1 - b
                pltpu.make_async_copy(
                    xrm_hbm.at[gidx[b]], bufs[b], gsem[b]).wait()

                def sub(jb, _):
                    base = cb + jb * L
                    wv16 = pw[pl.ds(base, L)]
                    rv16 = prel[pl.ds(base, L)]
                    for i in range(L):
                        w_b = jnp.full((L,), wv16[i], jnp.float32)
                        r_e = rv16[i]
                        e = jb * L + i
                        for g in range(G):
                            xv = bufs[b][e, pl.ds(g * L, L)]
                            ev = em_v[r_e, pl.ds(g * L, L)]
                            bufs[b][e, pl.ds(g * L, L)] = jnp.maximum(
                                xv + w_b * ev, 0.0)
                    return 0
                lax.fori_loop(0, CA // L, sub, 0)

                thr = lax.min(c * CA, EPW) - cb
                for j in range(CA // L):
                    v = pdst[pl.ds(cb + j * L, L)]
                    lanes = j * L + lax.iota(jnp.int32, L)
                    sdix[b][pl.ds(j * L, L)] = jnp.where(
                        lanes >= thr, v, jnp.full((L,), N, jnp.int32))

                pltpu.async_copy(bufs[b], agg_sh.at[sdix[b]], ssem[b],
                                 add=True)

                @pl.when(c + 1 < NCHA)
                def _pf():
                    @pl.when(c >= 1)
                    def _drain():
                        pltpu.make_async_copy(
                            bufs[ob], agg_sh.at[sdix[ob]], ssem[ob]).wait()
                    cb1 = lax.min((c + 1) * CA, EPW - CA)
                    pltpu.sync_copy(src_hbm.at[pl.ds(eb + cb1, CA)], gidx[ob])
                    pltpu.async_copy(xrm_hbm.at[gidx[ob]], bufs[ob], gsem[ob])
            return 0

        lax.fori_loop(0, NCHA // 2, body, 0)
        pltpu.make_async_copy(bufs[0], agg_sh.at[sdix[0]], ssem[0]).wait()
        pltpu.make_async_copy(bufs[1], agg_sh.at[sdix[1]], ssem[1]).wait()
        plsc.subcore_barrier()

        base_o = sid * ZB

        @pl.when(sid < NS - 1)
        def _w_full():
            pltpu.sync_copy(agg_sh.at[pl.ds(base_o, ZB)],
                            out_hbm.at[cid, pl.ds(base_o, ZB)])

        @pl.when(sid == NS - 1)
        def _w_last():
            rows = N - (NS - 1) * ZB
            pltpu.sync_copy(agg_sh.at[pl.ds(base_o, rows)],
                            out_hbm.at[cid, pl.ds(base_o, rows)])

    return k(xrm, src, dst, rel, w)


def _sc_triples(xout, rel_emb, src, dst, rel, w):
    mesh = plsc.VectorSubcoreMesh(core_axis_name="c", subcore_axis_name="s")

    @functools.partial(
        pl.kernel,
        out_type=jax.ShapeDtypeStruct((3 * E, D), jnp.float32),
        mesh=mesh,
        scratch_types=[
            pltpu.VMEM((R, D), jnp.float32),
            pltpu.VMEM((EPW,), jnp.int32),
            pltpu.VMEM((EPW,), jnp.int32),
            pltpu.VMEM((EPW,), jnp.int32),
            pltpu.VMEM((EPW,), jnp.float32),
            [pltpu.VMEM((C, D), jnp.float32) for _ in range(2)],
            [pltpu.VMEM((C, D), jnp.float32) for _ in range(2)],
            [pltpu.VMEM((C, D), jnp.float32) for _ in range(2)],
            [pltpu.VMEM((C,), jnp.int32) for _ in range(2)],
            [pltpu.VMEM((C,), jnp.int32) for _ in range(2)],
            [pltpu.VMEM((C,), jnp.int32) for _ in range(2)],
            [pltpu.VMEM((C,), jnp.int32) for _ in range(2)],
            [pltpu.VMEM((C,), jnp.int32) for _ in range(2)],
            [pltpu.SemaphoreType.DMA for _ in range(2)],
            [pltpu.SemaphoreType.DMA for _ in range(2)],
            [pltpu.SemaphoreType.DMA for _ in range(2)],
            [pltpu.SemaphoreType.DMA for _ in range(2)],
            [pltpu.SemaphoreType.DMA for _ in range(2)],
        ],
    )
    def k(xout_hbm, emr_hbm, src_hbm, dst_hbm, rel_hbm, w_hbm, out_hbm,
          emr_v, psrc, pdst, prel, pw, bufs, bufd, bufa, gis, gid,
          oxs, oxa, oxd, gss, gsd, sss, ssa, ssd):
        cid = lax.axis_index("c")
        sid = lax.axis_index("s")
        wid = sid * NC + cid
        eb = wid * EPW
        pltpu.sync_copy(src_hbm.at[pl.ds(eb, EPW)], psrc)
        pltpu.sync_copy(dst_hbm.at[pl.ds(eb, EPW)], pdst)
        pltpu.sync_copy(rel_hbm.at[pl.ds(eb, EPW)], prel)
        pltpu.sync_copy(w_hbm.at[pl.ds(eb, EPW)], pw)
        pltpu.sync_copy(emr_hbm, emr_v)
        gbase = eb * 3

        def fill_gidx(bb, cb):
            for j in range(C // L):
                gis[bb][pl.ds(j * L, L)] = psrc[pl.ds(cb + j * L, L)]
                gid[bb][pl.ds(j * L, L)] = pdst[pl.ds(cb + j * L, L)]

        def issue_gathers(bb):
            pltpu.async_copy(xout_hbm.at[gis[bb]], bufs[bb], gss[bb])
            pltpu.async_copy(xout_hbm.at[gid[bb]], bufd[bb], gsd[bb])

        def drain_scatters(bb):
            pltpu.make_async_copy(bufs[bb], out_hbm.at[oxs[bb]],
                                  sss[bb]).wait()
            pltpu.make_async_copy(bufa[bb], out_hbm.at[oxa[bb]],
                                  ssa[bb]).wait()
            pltpu.make_async_copy(bufd[bb], out_hbm.at[oxd[bb]],
                                  ssd[bb]).wait()

        fill_gidx(0, 0)
        issue_gathers(0)

        def body(k2, _):
            for b in range(2):
                c = k2 * 2 + b
                cb = lax.min(c * C, LAST_BASE)
                ob = 1 - b
                pltpu.make_async_copy(
                    xout_hbm.at[gis[b]], bufs[b], gss[b]).wait()
                pltpu.make_async_copy(
                    xout_hbm.at[gid[b]], bufd[b], gsd[b]).wait()

                def sub(jb, _):
                    base = cb + jb * L
                    wv16 = pw[pl.ds(base, L)]
                    rv16 = prel[pl.ds(base, L)]
                    for i in range(L):
                        w_b = jnp.full((L,), wv16[i], jnp.float32)
                        r_e = rv16[i]
                        e = jb * L + i
                        for g in range(G):
                            bufa[b][e, pl.ds(g * L, L)] = (
                                w_b * emr_v[r_e, pl.ds(g * L, L)])
                    return 0
                lax.fori_loop(0, C // L, sub, 0)

                for j in range(C // L):
                    rows3 = (gbase + (cb + j * L + lax.iota(jnp.int32, L)) * 3)
                    oxs[b][pl.ds(j * L, L)] = rows3
                    oxa[b][pl.ds(j * L, L)] = rows3 + 1
                    oxd[b][pl.ds(j * L, L)] = rows3 + 2

                pltpu.async_copy(bufs[b], out_hbm.at[oxs[b]], sss[b])
                pltpu.async_copy(bufa[b], out_hbm.at[oxa[b]], ssa[b])
                pltpu.async_copy(bufd[b], out_hbm.at[oxd[b]], ssd[b])

                @pl.when(c + 1 < NCHUNK)
                def _pf():
                    @pl.when(c >= 1)
                    def _drain():
                        drain_scatters(ob)
                    cb1 = lax.min((c + 1) * C, LAST_BASE)
                    fill_gidx(ob, cb1)
                    issue_gathers(ob)
            return 0

        lax.fori_loop(0, NCHUNK // 2, body, 0)
        drain_scatters(0)
        drain_scatters(1)

    return k(xout, rel_emb, src, dst, rel, w)


def kernel(concept_embedding, relation_embedding, edge_index, edge_relation,
           edge_weight, W_msg, W_self, W_upd):
    x = concept_embedding
    src = edge_index[0]
    dst = edge_index[1]
    xr = jnp.concatenate(
        [x, relation_embedding,
         jnp.zeros((10240 - N - R, D), jnp.float32)], axis=0)
    xrm = _tc_matmul(xr, W_msg, bm=512)
    aggp = _sc_agg(xrm, src, dst, edge_relation, edge_weight)
    xout = _tc_update(aggp, x, W_upd, W_self)
    trip = _sc_triples(xout, relation_embedding, src, dst,
                       edge_relation, edge_weight)
    return trip.reshape(E, 3 * D)


# speedup vs baseline: 3.6160x; 1.4273x over previous
import functools

import jax
import jax.numpy as jnp
from jax import lax
from jax.experimental import pallas as pl
from jax.experimental.pallas import tpu as pltpu
from jax.experimental.pallas import tpu_sc as plsc

N = 10000
E = 160000
D = 128
R = 40

NC, NS, L = 2, 16, 16
NW = NC * NS
EPW = E // NW
C = 128
NCH = 40
CLAMP = EPW - C
G = D // L
ZB = 632
NPAD = NS * ZB

EBM = 640
NEB = E // EBM


def _mm_body(a_ref, w_ref, o_ref):
    o_ref[...] = jnp.dot(a_ref[...], w_ref[...],
                         preferred_element_type=jnp.float32)


def _tc_matmul(a, w, bm):
    m = a.shape[0]
    return pl.pallas_call(
        _mm_body,
        grid=(m // bm,),
        in_specs=[pl.BlockSpec((bm, D), lambda i: (i, 0)),
                  pl.BlockSpec((D, D), lambda i: (0, 0))],
        out_specs=pl.BlockSpec((bm, D), lambda i: (i, 0)),
        out_shape=jax.ShapeDtypeStruct((m, D), jnp.float32),
    )(a, w)


def _attr_body(rel_ref, w_ref, remb_ref, wmsg_ref, ea_ref, eam_ref):
    relb = rel_ref[0, 0]
    wb = w_ref[0, 0]
    onehot = (relb[:, None]
              == lax.broadcasted_iota(jnp.int32, (EBM, R), 1))
    ea = wb[:, None] * jnp.dot(onehot.astype(jnp.float32), remb_ref[...],
                               preferred_element_type=jnp.float32)
    ea_ref[...] = ea
    eam_ref[...] = jnp.dot(ea, wmsg_ref[...],
                           preferred_element_type=jnp.float32)


def _tc_edge_attr(rel2, w2, rel_emb, w_msg):
    return pl.pallas_call(
        _attr_body,
        grid=(NEB,),
        in_specs=[pl.BlockSpec((1, 1, EBM), lambda i: (i, 0, 0)),
                  pl.BlockSpec((1, 1, EBM), lambda i: (i, 0, 0)),
                  pl.BlockSpec((R, D), lambda i: (0, 0)),
                  pl.BlockSpec((D, D), lambda i: (0, 0))],
        out_specs=[pl.BlockSpec((EBM, D), lambda i: (i, 0)),
                   pl.BlockSpec((EBM, D), lambda i: (i, 0))],
        out_shape=[jax.ShapeDtypeStruct((E, D), jnp.float32),
                   jax.ShapeDtypeStruct((E, D), jnp.float32)],
    )(rel2, w2, rel_emb, w_msg)


def _upd_body(agg_ref, x_ref, wu_ref, ws_ref, o_ref):
    a = agg_ref[0] + agg_ref[1]
    o_ref[...] = jnp.maximum(
        jnp.dot(a, wu_ref[...], preferred_element_type=jnp.float32)
        + jnp.dot(x_ref[...], ws_ref[...], preferred_element_type=jnp.float32),
        0.0)


def _tc_update(aggp, x, w_upd, w_self, bm=400):
    return pl.pallas_call(
        _upd_body,
        grid=(N // bm,),
        in_specs=[pl.BlockSpec((NC, bm, D), lambda i: (0, i, 0)),
                  pl.BlockSpec((bm, D), lambda i: (i, 0)),
                  pl.BlockSpec((D, D), lambda i: (0, 0)),
                  pl.BlockSpec((D, D), lambda i: (0, 0))],
        out_specs=pl.BlockSpec((bm, D), lambda i: (i, 0)),
        out_shape=jax.ShapeDtypeStruct((N, D), jnp.float32),
    )(aggp, x, w_upd, w_self)


def _sc_agg(xm, eam, src, dst):
    mesh = plsc.VectorSubcoreMesh(core_axis_name="c", subcore_axis_name="s")

    @functools.partial(
        pl.kernel,
        out_type=jax.ShapeDtypeStruct((NC, N, D), jnp.float32),
        mesh=mesh,
        scratch_types=[
            pltpu.VMEM((EPW,), jnp.int32),
            pltpu.VMEM((EPW,), jnp.int32),
            [pltpu.VMEM((C, D), jnp.float32) for _ in range(2)],
            [pltpu.VMEM((C,), jnp.int32) for _ in range(2)],
            [pltpu.VMEM((C,), jnp.int32) for _ in range(2)],
            [pltpu.VMEM((C,), jnp.int32) for _ in range(2)],
            [pltpu.SemaphoreType.DMA for _ in range(2)],
            [pltpu.SemaphoreType.DMA for _ in range(2)],
            [pltpu.SemaphoreType.DMA for _ in range(2)],
            pltpu.VMEM_SHARED((NPAD, D), jnp.float32),
        ],
    )
    def k(xm_hbm, eam_hbm, src_hbm, dst_hbm, out_hbm,
          psrc, pdst, bufs, gidx, lidx, sdix, gsem, asem, ssem, agg_sh):
        buf = bufs[0]
        cid = lax.axis_index("c")
        sid = lax.axis_index("s")
        wid = sid * NC + cid

        def zrow(i, _):
            r = i // G
            g = i % G
            buf[r, pl.ds(g * L, L)] = jnp.zeros((L,), jnp.float32)
            return 0
        lax.fori_loop(0, C * G, zrow, 0)
        base_z = sid * ZB
        for b in range(ZB // C):
            pltpu.sync_copy(buf, agg_sh.at[pl.ds(base_z + b * C, C)])
        pltpu.sync_copy(buf.at[pl.ds(0, ZB % C)],
                        agg_sh.at[pl.ds(base_z + (ZB // C) * C, ZB % C)])
        plsc.subcore_barrier()

        eb = wid * EPW
        pltpu.sync_copy(src_hbm.at[pl.ds(eb, EPW)], psrc)
        pltpu.sync_copy(dst_hbm.at[pl.ds(eb, EPW)], pdst)

        def fill_idx(bb, cb):
            for j in range(C // L):
                lane = lax.iota(jnp.int32, L)
                gidx[bb][pl.ds(j * L, L)] = psrc[pl.ds(cb + j * L, L)]
                lidx[bb][pl.ds(j * L, L)] = eb + cb + j * L + lane

        fill_idx(0, 0)
        pltpu.async_copy(xm_hbm.at[gidx[0]], bufs[0], gsem[0])

        def body(k2, _):
            for b in range(2):
                c = k2 * 2 + b
                cb = lax.min(c * C, CLAMP)
                ob = 1 - b
                pltpu.make_async_copy(
                    xm_hbm.at[gidx[b]], bufs[b], gsem[b]).wait()
                pltpu.async_copy(eam_hbm.at[lidx[b]], bufs[b], asem[b],
                                 add=True)

                thr = lax.min(c * C, EPW) - cb
                for j in range(C // L):
                    v = pdst[pl.ds(cb + j * L, L)]
                    lanes = j * L + lax.iota(jnp.int32, L)
                    sdix[b][pl.ds(j * L, L)] = jnp.where(
                        lanes >= thr, v, jnp.full((L,), N, jnp.int32))

                @pl.when(c + 1 < NCH)
                def _pf():
                    @pl.when(c >= 1)
                    def _drain():
                        pltpu.make_async_copy(
                            bufs[ob], agg_sh.at[sdix[ob]], ssem[ob]).wait()
                    cb1 = lax.min((c + 1) * C, CLAMP)
                    fill_idx(ob, cb1)
                    pltpu.async_copy(xm_hbm.at[gidx[ob]], bufs[ob], gsem[ob])

                pltpu.make_async_copy(
                    eam_hbm.at[lidx[b]], bufs[b], asem[b]).wait()

                def sub(i, _):
                    r = i // G
                    g = i % G
                    v = bufs[b][r, pl.ds(g * L, L)]
                    bufs[b][r, pl.ds(g * L, L)] = jnp.maximum(v, 0.0)
                    return 0
                lax.fori_loop(0, C * G, sub, 0)

                pltpu.async_copy(bufs[b], agg_sh.at[sdix[b]], ssem[b],
                                 add=True)
            return 0

        lax.fori_loop(0, NCH // 2, body, 0)
        pltpu.make_async_copy(bufs[0], agg_sh.at[sdix[0]], ssem[0]).wait()
        pltpu.make_async_copy(bufs[1], agg_sh.at[sdix[1]], ssem[1]).wait()
        plsc.subcore_barrier()

        base_o = sid * ZB

        @pl.when(sid < NS - 1)
        def _w_full():
            pltpu.sync_copy(agg_sh.at[pl.ds(base_o, ZB)],
                            out_hbm.at[cid, pl.ds(base_o, ZB)])

        @pl.when(sid == NS - 1)
        def _w_last():
            rows = N - (NS - 1) * ZB
            pltpu.sync_copy(agg_sh.at[pl.ds(base_o, rows)],
                            out_hbm.at[cid, pl.ds(base_o, rows)])

    return k(xm, eam, src, dst)


def _sc_triples(xout, ea, src, dst):
    mesh = plsc.VectorSubcoreMesh(core_axis_name="c", subcore_axis_name="s")

    @functools.partial(
        pl.kernel,
        out_type=jax.ShapeDtypeStruct((E, 3 * D), jnp.float32),
        mesh=mesh,
        scratch_types=[
            pltpu.VMEM((EPW,), jnp.int32),
            pltpu.VMEM((EPW,), jnp.int32),
            [pltpu.VMEM((C, 3 * D), jnp.float32) for _ in range(2)],
            [pltpu.VMEM((C,), jnp.int32) for _ in range(2)],
            [pltpu.VMEM((C,), jnp.int32) for _ in range(2)],
            [pltpu.VMEM((C,), jnp.int32) for _ in range(2)],
            [pltpu.SemaphoreType.DMA for _ in range(2)],
            [pltpu.SemaphoreType.DMA for _ in range(2)],
            [pltpu.SemaphoreType.DMA for _ in range(2)],
            [pltpu.SemaphoreType.DMA for _ in range(2)],
        ],
    )
    def k(xout_hbm, ea_hbm, src_hbm, dst_hbm, out_hbm,
          psrc, pdst, ivl, gis, gid, oxe, gss, gse, gsd, ssc):
        cid = lax.axis_index("c")
        sid = lax.axis_index("s")
        wid = sid * NC + cid
        eb = wid * EPW
        pltpu.sync_copy(src_hbm.at[pl.ds(eb, EPW)], psrc)
        pltpu.sync_copy(dst_hbm.at[pl.ds(eb, EPW)], pdst)

        def fill_idx(bb, cb):
            for j in range(C // L):
                lane = lax.iota(jnp.int32, L)
                gis[bb][pl.ds(j * L, L)] = psrc[pl.ds(cb + j * L, L)]
                gid[bb][pl.ds(j * L, L)] = pdst[pl.ds(cb + j * L, L)]
                oxe[bb][pl.ds(j * L, L)] = eb + cb + j * L + lane

        def issue_gathers(bb):
            pltpu.async_copy(xout_hbm.at[gis[bb]],
                             ivl[bb].at[:, pl.ds(0, D)], gss[bb])
            pltpu.async_copy(ea_hbm.at[oxe[bb]],
                             ivl[bb].at[:, pl.ds(D, D)], gse[bb])
            pltpu.async_copy(xout_hbm.at[gid[bb]],
                             ivl[bb].at[:, pl.ds(2 * D, D)], gsd[bb])

        def wait_gathers(bb):
            pltpu.make_async_copy(xout_hbm.at[gis[bb]],
                                  ivl[bb].at[:, pl.ds(0, D)], gss[bb]).wait()
            pltpu.make_async_copy(ea_hbm.at[oxe[bb]],
                                  ivl[bb].at[:, pl.ds(D, D)], gse[bb]).wait()
            pltpu.make_async_copy(xout_hbm.at[gid[bb]],
                                  ivl[bb].at[:, pl.ds(2 * D, D)],
                                  gsd[bb]).wait()

        fill_idx(0, 0)
        issue_gathers(0)

        def body(k2, _):
            for b in range(2):
                c = k2 * 2 + b
                ob = 1 - b
                wait_gathers(b)
                pltpu.async_copy(ivl[b], out_hbm.at[oxe[b]], ssc[b])

                @pl.when(c + 1 < NCH)
                def _pf():
                    @pl.when(c >= 1)
                    def _drain():
                        pltpu.make_async_copy(
                            ivl[ob], out_hbm.at[oxe[ob]], ssc[ob]).wait()
                    cb1 = lax.min((c + 1) * C, CLAMP)
                    fill_idx(ob, cb1)
                    issue_gathers(ob)
            return 0

        lax.fori_loop(0, NCH // 2, body, 0)
        pltpu.make_async_copy(ivl[0], out_hbm.at[oxe[0]], ssc[0]).wait()
        pltpu.make_async_copy(ivl[1], out_hbm.at[oxe[1]], ssc[1]).wait()

    return k(xout, ea, src, dst)


def kernel(concept_embedding, relation_embedding, edge_index, edge_relation,
           edge_weight, W_msg, W_self, W_upd):
    x = concept_embedding
    src = edge_index[0]
    dst = edge_index[1]
    rel2 = edge_relation.reshape(NEB, 1, EBM)
    w2 = edge_weight.reshape(NEB, 1, EBM)
    ea, eam = _tc_edge_attr(rel2, w2, relation_embedding, W_msg)
    xm = _tc_matmul(x, W_msg, bm=400)
    aggp = _sc_agg(xm, eam, src, dst)
    xout = _tc_update(aggp, x, W_upd, W_self)
    return _sc_triples(xout, ea, src, dst)


# speedup vs baseline: 5.5829x; 1.5439x over previous
import functools

import jax
import jax.numpy as jnp
from jax import lax
from jax.experimental import pallas as pl
from jax.experimental.pallas import tpu as pltpu
from jax.experimental.pallas import tpu_sc as plsc

N = 10000
E = 160000
D = 128
R = 40

NC, NS, L = 2, 16, 16
NW = NC * NS
EPW = E // NW
C = 128
NCH = 40
CLAMP = EPW - C
G = D // L
ZB = 632
NPAD = NS * ZB

EBM = 6400
NEB = E // EBM


def _mm_body(a_ref, w_ref, o_ref):
    o_ref[...] = jnp.dot(a_ref[...], w_ref[...],
                         preferred_element_type=jnp.float32)


def _tc_matmul(a, w, bm):
    m = a.shape[0]
    return pl.pallas_call(
        _mm_body,
        grid=(m // bm,),
        in_specs=[pl.BlockSpec((bm, D), lambda i: (i, 0)),
                  pl.BlockSpec((D, D), lambda i: (0, 0))],
        out_specs=pl.BlockSpec((bm, D), lambda i: (i, 0)),
        out_shape=jax.ShapeDtypeStruct((m, D), jnp.float32),
    )(a, w)


def _attr_body(rel_ref, w_ref, remb_ref, wmsg_ref, ea_ref, eam_ref):
    relb = rel_ref[0, 0]
    wb = w_ref[0, 0]
    onehot = (relb[:, None]
              == lax.broadcasted_iota(jnp.int32, (EBM, R), 1))
    ea = wb[:, None] * jnp.dot(onehot.astype(jnp.float32), remb_ref[...],
                               preferred_element_type=jnp.float32)
    ea_ref[...] = ea
    eam_ref[...] = jnp.dot(ea, wmsg_ref[...],
                           preferred_element_type=jnp.float32)


def _tc_edge_attr(rel2, w2, rel_emb, w_msg):
    return pl.pallas_call(
        _attr_body,
        grid=(NEB,),
        in_specs=[pl.BlockSpec((1, 1, EBM), lambda i: (i, 0, 0)),
                  pl.BlockSpec((1, 1, EBM), lambda i: (i, 0, 0)),
                  pl.BlockSpec((R, D), lambda i: (0, 0)),
                  pl.BlockSpec((D, D), lambda i: (0, 0))],
        out_specs=[pl.BlockSpec((EBM, D), lambda i: (i, 0)),
                   pl.BlockSpec((EBM, D), lambda i: (i, 0))],
        out_shape=[jax.ShapeDtypeStruct((E, D), jnp.float32),
                   jax.ShapeDtypeStruct((E, D), jnp.float32)],
    )(rel2, w2, rel_emb, w_msg)


def _upd_body(agg_ref, x_ref, wu_ref, ws_ref, o_ref):
    a = agg_ref[0] + agg_ref[1]
    o_ref[...] = jnp.maximum(
        jnp.dot(a, wu_ref[...], preferred_element_type=jnp.float32)
        + jnp.dot(x_ref[...], ws_ref[...], preferred_element_type=jnp.float32),
        0.0)


def _tc_update(aggp, x, w_upd, w_self, bm=400):
    return pl.pallas_call(
        _upd_body,
        grid=(N // bm,),
        in_specs=[pl.BlockSpec((NC, bm, D), lambda i: (0, i, 0)),
                  pl.BlockSpec((bm, D), lambda i: (i, 0)),
                  pl.BlockSpec((D, D), lambda i: (0, 0)),
                  pl.BlockSpec((D, D), lambda i: (0, 0))],
        out_specs=pl.BlockSpec((bm, D), lambda i: (i, 0)),
        out_shape=jax.ShapeDtypeStruct((N, D), jnp.float32),
    )(aggp, x, w_upd, w_self)


def _sc_agg(xm, eam, src, dst):
    mesh = plsc.VectorSubcoreMesh(core_axis_name="c", subcore_axis_name="s")

    @functools.partial(
        pl.kernel,
        out_type=jax.ShapeDtypeStruct((NC, N, D), jnp.float32),
        mesh=mesh,
        scratch_types=[
            pltpu.VMEM((EPW,), jnp.int32),
            pltpu.VMEM((EPW,), jnp.int32),
            [pltpu.VMEM((C, D), jnp.float32) for _ in range(2)],
            [pltpu.VMEM((C,), jnp.int32) for _ in range(2)],
            [pltpu.VMEM((C,), jnp.int32) for _ in range(2)],
            [pltpu.VMEM((C,), jnp.int32) for _ in range(2)],
            [pltpu.SemaphoreType.DMA for _ in range(2)],
            [pltpu.SemaphoreType.DMA for _ in range(2)],
            [pltpu.SemaphoreType.DMA for _ in range(2)],
            pltpu.VMEM_SHARED((NPAD, D), jnp.float32),
        ],
    )
    def k(xm_hbm, eam_hbm, src_hbm, dst_hbm, out_hbm,
          psrc, pdst, bufs, gidx, lidx, sdix, gsem, asem, ssem, agg_sh):
        buf = bufs[0]
        cid = lax.axis_index("c")
        sid = lax.axis_index("s")
        wid = sid * NC + cid

        def zrow(i, _):
            r = i // G
            g = i % G
            buf[r, pl.ds(g * L, L)] = jnp.zeros((L,), jnp.float32)
            return 0
        lax.fori_loop(0, C * G, zrow, 0)
        base_z = sid * ZB
        for b in range(ZB // C):
            pltpu.sync_copy(buf, agg_sh.at[pl.ds(base_z + b * C, C)])
        pltpu.sync_copy(buf.at[pl.ds(0, ZB % C)],
                        agg_sh.at[pl.ds(base_z + (ZB // C) * C, ZB % C)])
        plsc.subcore_barrier()

        eb = wid * EPW
        pltpu.sync_copy(src_hbm.at[pl.ds(eb, EPW)], psrc)
        pltpu.sync_copy(dst_hbm.at[pl.ds(eb, EPW)], pdst)

        def fill_idx(bb, cb):
            for j in range(C // L):
                lane = lax.iota(jnp.int32, L)
                gidx[bb][pl.ds(j * L, L)] = psrc[pl.ds(cb + j * L, L)]
                lidx[bb][pl.ds(j * L, L)] = eb + cb + j * L + lane

        def zero_buf(bb):
            def zb(r, _):
                for g in range(G):
                    bufs[bb][r, pl.ds(g * L, L)] = jnp.zeros((L,), jnp.float32)
                return 0
            lax.fori_loop(0, C, zb, 0)

        def issue_adds(bb):
            pltpu.async_copy(xm_hbm.at[gidx[bb]], bufs[bb], gsem[bb],
                             add=True)
            pltpu.async_copy(eam_hbm.at[lidx[bb]], bufs[bb], asem[bb],
                             add=True)

        def wait_adds(bb):
            pltpu.make_async_copy(
                xm_hbm.at[gidx[bb]], bufs[bb], gsem[bb]).wait()
            pltpu.make_async_copy(
                eam_hbm.at[lidx[bb]], bufs[bb], asem[bb]).wait()

        fill_idx(0, 0)
        issue_adds(0)

        def body(k2, _):
            for b in range(2):
                c = k2 * 2 + b
                cb = lax.min(c * C, CLAMP)
                ob = 1 - b
                wait_adds(b)

                thr = lax.min(c * C, EPW) - cb
                for j in range(C // L):
                    v = pdst[pl.ds(cb + j * L, L)]
                    lanes = j * L + lax.iota(jnp.int32, L)
                    sdix[b][pl.ds(j * L, L)] = jnp.where(
                        lanes >= thr, v, jnp.full((L,), N, jnp.int32))

                def sub(r, _):
                    for g in range(G):
                        v = bufs[b][r, pl.ds(g * L, L)]
                        bufs[b][r, pl.ds(g * L, L)] = jnp.maximum(v, 0.0)
                    return 0
                lax.fori_loop(0, C, sub, 0)

                pltpu.async_copy(bufs[b], agg_sh.at[sdix[b]], ssem[b],
                                 add=True)

                @pl.when(c + 1 < NCH)
                def _pf():
                    @pl.when(c >= 1)
                    def _drain():
                        pltpu.make_async_copy(
                            bufs[ob], agg_sh.at[sdix[ob]], ssem[ob]).wait()
                    zero_buf(ob)
                    cb1 = lax.min((c + 1) * C, CLAMP)
                    fill_idx(ob, cb1)
                    issue_adds(ob)
            return 0

        lax.fori_loop(0, NCH // 2, body, 0)
        pltpu.make_async_copy(bufs[0], agg_sh.at[sdix[0]], ssem[0]).wait()
        pltpu.make_async_copy(bufs[1], agg_sh.at[sdix[1]], ssem[1]).wait()
        plsc.subcore_barrier()

        base_o = sid * ZB

        @pl.when(sid < NS - 1)
        def _w_full():
            pltpu.sync_copy(agg_sh.at[pl.ds(base_o, ZB)],
                            out_hbm.at[cid, pl.ds(base_o, ZB)])

        @pl.when(sid == NS - 1)
        def _w_last():
            rows = N - (NS - 1) * ZB
            pltpu.sync_copy(agg_sh.at[pl.ds(base_o, rows)],
                            out_hbm.at[cid, pl.ds(base_o, rows)])

    return k(xm, eam, src, dst)


def _sc_triples(xout, ea, src, dst):
    mesh = plsc.VectorSubcoreMesh(core_axis_name="c", subcore_axis_name="s")

    @functools.partial(
        pl.kernel,
        out_type=jax.ShapeDtypeStruct((E, 3 * D), jnp.float32),
        mesh=mesh,
        scratch_types=[
            pltpu.VMEM((EPW,), jnp.int32),
            pltpu.VMEM((EPW,), jnp.int32),
            [pltpu.VMEM((C, 3 * D), jnp.float32) for _ in range(2)],
            [pltpu.VMEM((C,), jnp.int32) for _ in range(2)],
            [pltpu.VMEM((C,), jnp.int32) for _ in range(2)],
            [pltpu.VMEM((C,), jnp.int32) for _ in range(2)],
            [pltpu.SemaphoreType.DMA for _ in range(2)],
            [pltpu.SemaphoreType.DMA for _ in range(2)],
            [pltpu.SemaphoreType.DMA for _ in range(2)],
            [pltpu.SemaphoreType.DMA for _ in range(2)],
        ],
    )
    def k(xout_hbm, ea_hbm, src_hbm, dst_hbm, out_hbm,
          psrc, pdst, ivl, gis, gid, oxe, gss, gse, gsd, ssc):
        cid = lax.axis_index("c")
        sid = lax.axis_index("s")
        wid = sid * NC + cid
        eb = wid * EPW
        pltpu.sync_copy(src_hbm.at[pl.ds(eb, EPW)], psrc)
        pltpu.sync_copy(dst_hbm.at[pl.ds(eb, EPW)], pdst)

        def fill_idx(bb, cb):
            for j in range(C // L):
                lane = lax.iota(jnp.int32, L)
                gis[bb][pl.ds(j * L, L)] = psrc[pl.ds(cb + j * L, L)]
                gid[bb][pl.ds(j * L, L)] = pdst[pl.ds(cb + j * L, L)]
                oxe[bb][pl.ds(j * L, L)] = eb + cb + j * L + lane

        def issue_gathers(bb):
            pltpu.async_copy(xout_hbm.at[gis[bb]],
                             ivl[bb].at[:, pl.ds(0, D)], gss[bb])
            pltpu.async_copy(ea_hbm.at[oxe[bb]],
                             ivl[bb].at[:, pl.ds(D, D)], gse[bb])
            pltpu.async_copy(xout_hbm.at[gid[bb]],
                             ivl[bb].at[:, pl.ds(2 * D, D)], gsd[bb])

        def wait_gathers(bb):
            pltpu.make_async_copy(xout_hbm.at[gis[bb]],
                                  ivl[bb].at[:, pl.ds(0, D)], gss[bb]).wait()
            pltpu.make_async_copy(ea_hbm.at[oxe[bb]],
                                  ivl[bb].at[:, pl.ds(D, D)], gse[bb]).wait()
            pltpu.make_async_copy(xout_hbm.at[gid[bb]],
                                  ivl[bb].at[:, pl.ds(2 * D, D)],
                                  gsd[bb]).wait()

        fill_idx(0, 0)
        issue_gathers(0)

        def body(k2, _):
            for b in range(2):
                c = k2 * 2 + b
                ob = 1 - b
                wait_gathers(b)
                pltpu.async_copy(ivl[b], out_hbm.at[oxe[b]], ssc[b])

                @pl.when(c + 1 < NCH)
                def _pf():
                    @pl.when(c >= 1)
                    def _drain():
                        pltpu.make_async_copy(
                            ivl[ob], out_hbm.at[oxe[ob]], ssc[ob]).wait()
                    cb1 = lax.min((c + 1) * C, CLAMP)
                    fill_idx(ob, cb1)
                    issue_gathers(ob)
            return 0

        lax.fori_loop(0, NCH // 2, body, 0)
        pltpu.make_async_copy(ivl[0], out_hbm.at[oxe[0]], ssc[0]).wait()
        pltpu.make_async_copy(ivl[1], out_hbm.at[oxe[1]], ssc[1]).wait()

    return k(xout, ea, src, dst)


def kernel(concept_embedding, relation_embedding, edge_index, edge_relation,
           edge_weight, W_msg, W_self, W_upd):
    x = concept_embedding
    src = edge_index[0]
    dst = edge_index[1]
    rel2 = edge_relation.reshape(NEB, 1, EBM)
    w2 = edge_weight.reshape(NEB, 1, EBM)
    ea, eam = _tc_edge_attr(rel2, w2, relation_embedding, W_msg)
    xm = _tc_matmul(x, W_msg, bm=400)
    aggp = _sc_agg(xm, eam, src, dst)
    xout = _tc_update(aggp, x, W_upd, W_self)
    return _sc_triples(xout, ea, src, dst)


# speedup vs baseline: 5.7349x; 1.0272x over previous
import functools

import jax
import jax.numpy as jnp
from jax import lax
from jax.experimental import pallas as pl
from jax.experimental.pallas import tpu as pltpu
from jax.experimental.pallas import tpu_sc as plsc

N = 10000
E = 160000
D = 128
R = 40

NC, NS, L = 2, 16, 16
NW = NC * NS
EPW = E // NW
C = 128
NCH = 40
CLAMP = EPW - C
G = D // L
ZB = 632
NPAD = NS * ZB

EBM = 6400
NEB = E // EBM


def _mm_body(a_ref, w_ref, o_ref):
    o_ref[...] = jnp.dot(a_ref[...], w_ref[...],
                         preferred_element_type=jnp.float32)


def _tc_matmul(a, w, bm):
    m = a.shape[0]
    return pl.pallas_call(
        _mm_body,
        grid=(m // bm,),
        in_specs=[pl.BlockSpec((bm, D), lambda i: (i, 0)),
                  pl.BlockSpec((D, D), lambda i: (0, 0))],
        out_specs=pl.BlockSpec((bm, D), lambda i: (i, 0)),
        out_shape=jax.ShapeDtypeStruct((m, D), jnp.float32),
    )(a, w)


def _attr_body(rel_ref, w_ref, tab_ref, o_ref):
    relb = rel_ref[0, 0]
    wb = w_ref[0, 0]
    onehot = (relb[:, None]
              == lax.broadcasted_iota(jnp.int32, (EBM, R), 1))
    o_ref[...] = wb[:, None] * jnp.dot(
        onehot.astype(jnp.float32), tab_ref[...],
        preferred_element_type=jnp.float32)


def _tc_edge_attr(rel2, w2, tab):
    return pl.pallas_call(
        _attr_body,
        grid=(NEB,),
        in_specs=[pl.BlockSpec((1, 1, EBM), lambda i: (i, 0, 0)),
                  pl.BlockSpec((1, 1, EBM), lambda i: (i, 0, 0)),
                  pl.BlockSpec((R, D), lambda i: (0, 0))],
        out_specs=pl.BlockSpec((EBM, D), lambda i: (i, 0)),
        out_shape=jax.ShapeDtypeStruct((E, D), jnp.float32),
    )(rel2, w2, tab)


def _upd_body(agg_ref, x_ref, wu_ref, ws_ref, o_ref):
    a = agg_ref[0] + agg_ref[1]
    o_ref[...] = jnp.maximum(
        jnp.dot(a, wu_ref[...], preferred_element_type=jnp.float32)
        + jnp.dot(x_ref[...], ws_ref[...], preferred_element_type=jnp.float32),
        0.0)


def _tc_update(aggp, x, w_upd, w_self, bm=400):
    return pl.pallas_call(
        _upd_body,
        grid=(N // bm,),
        in_specs=[pl.BlockSpec((NC, bm, D), lambda i: (0, i, 0)),
                  pl.BlockSpec((bm, D), lambda i: (i, 0)),
                  pl.BlockSpec((D, D), lambda i: (0, 0)),
                  pl.BlockSpec((D, D), lambda i: (0, 0))],
        out_specs=pl.BlockSpec((bm, D), lambda i: (i, 0)),
        out_shape=jax.ShapeDtypeStruct((N, D), jnp.float32),
    )(aggp, x, w_upd, w_self)


def _sc_agg(xm, eam, eidx):
    mesh = plsc.VectorSubcoreMesh(core_axis_name="c", subcore_axis_name="s")

    @functools.partial(
        pl.kernel,
        out_type=jax.ShapeDtypeStruct((NC, N, D), jnp.float32),
        mesh=mesh,
        scratch_types=[
            pltpu.VMEM((EPW,), jnp.int32),
            pltpu.VMEM((EPW,), jnp.int32),
            [pltpu.VMEM((C, D), jnp.float32) for _ in range(2)],
            [pltpu.VMEM((C,), jnp.int32) for _ in range(2)],
            [pltpu.VMEM((C,), jnp.int32) for _ in range(2)],
            [pltpu.VMEM((C,), jnp.int32) for _ in range(2)],
            [pltpu.SemaphoreType.DMA for _ in range(2)],
            [pltpu.SemaphoreType.DMA for _ in range(2)],
            [pltpu.SemaphoreType.DMA for _ in range(2)],
            pltpu.VMEM_SHARED((NPAD, D), jnp.float32),
        ],
    )
    def k(xm_hbm, eam_hbm, eidx_hbm, out_hbm,
          psrc, pdst, bufs, gidx, lidx, sdix, gsem, asem, ssem, agg_sh):
        buf = bufs[0]
        cid = lax.axis_index("c")
        sid = lax.axis_index("s")
        wid = sid * NC + cid

        def zrow(i, _):
            r = i // G
            g = i % G
            buf[r, pl.ds(g * L, L)] = jnp.zeros((L,), jnp.float32)
            return 0
        lax.fori_loop(0, C * G, zrow, 0)
        base_z = sid * ZB
        for b in range(ZB // C):
            pltpu.sync_copy(buf, agg_sh.at[pl.ds(base_z + b * C, C)])
        pltpu.sync_copy(buf.at[pl.ds(0, ZB % C)],
                        agg_sh.at[pl.ds(base_z + (ZB // C) * C, ZB % C)])
        plsc.subcore_barrier()

        eb = wid * EPW
        pltpu.sync_copy(eidx_hbm.at[pl.ds(eb, EPW)], psrc)
        pltpu.sync_copy(eidx_hbm.at[pl.ds(E + eb, EPW)], pdst)

        def fill_idx(bb, cb):
            for j in range(C // L):
                lane = lax.iota(jnp.int32, L)
                gidx[bb][pl.ds(j * L, L)] = psrc[pl.ds(cb + j * L, L)]
                lidx[bb][pl.ds(j * L, L)] = eb + cb + j * L + lane

        def zero_buf(bb):
            def zb(r, _):
                for g in range(G):
                    bufs[bb][r, pl.ds(g * L, L)] = jnp.zeros((L,), jnp.float32)
                return 0
            lax.fori_loop(0, C, zb, 0)

        def issue_adds(bb):
            pltpu.async_copy(xm_hbm.at[gidx[bb]], bufs[bb], gsem[bb],
                             add=True)
            pltpu.async_copy(eam_hbm.at[lidx[bb]], bufs[bb], asem[bb],
                             add=True)

        def wait_adds(bb):
            pltpu.make_async_copy(
                xm_hbm.at[gidx[bb]], bufs[bb], gsem[bb]).wait()
            pltpu.make_async_copy(
                eam_hbm.at[lidx[bb]], bufs[bb], asem[bb]).wait()

        fill_idx(0, 0)
        issue_adds(0)

        def body(k2, _):
            for b in range(2):
                c = k2 * 2 + b
                cb = lax.min(c * C, CLAMP)
                ob = 1 - b
                wait_adds(b)

                thr = lax.min(c * C, EPW) - cb
                for j in range(C // L):
                    v = pdst[pl.ds(cb + j * L, L)]
                    lanes = j * L + lax.iota(jnp.int32, L)
                    sdix[b][pl.ds(j * L, L)] = jnp.where(
                        lanes >= thr, v, jnp.full((L,), N, jnp.int32))

                def sub(r, _):
                    for g in range(G):
                        v = bufs[b][r, pl.ds(g * L, L)]
                        bufs[b][r, pl.ds(g * L, L)] = jnp.maximum(v, 0.0)
                    return 0
                lax.fori_loop(0, C, sub, 0)

                pltpu.async_copy(bufs[b], agg_sh.at[sdix[b]], ssem[b],
                                 add=True)

                @pl.when(c + 1 < NCH)
                def _pf():
                    @pl.when(c >= 1)
                    def _drain():
                        pltpu.make_async_copy(
                            bufs[ob], agg_sh.at[sdix[ob]], ssem[ob]).wait()
                    zero_buf(ob)
                    cb1 = lax.min((c + 1) * C, CLAMP)
                    fill_idx(ob, cb1)
                    issue_adds(ob)
            return 0

        lax.fori_loop(0, NCH // 2, body, 0)
        pltpu.make_async_copy(bufs[0], agg_sh.at[sdix[0]], ssem[0]).wait()
        pltpu.make_async_copy(bufs[1], agg_sh.at[sdix[1]], ssem[1]).wait()
        plsc.subcore_barrier()

        base_o = sid * ZB

        @pl.when(sid < NS - 1)
        def _w_full():
            pltpu.sync_copy(agg_sh.at[pl.ds(base_o, ZB)],
                            out_hbm.at[cid, pl.ds(base_o, ZB)])

        @pl.when(sid == NS - 1)
        def _w_last():
            rows = N - (NS - 1) * ZB
            pltpu.sync_copy(agg_sh.at[pl.ds(base_o, rows)],
                            out_hbm.at[cid, pl.ds(base_o, rows)])

    return k(xm, eam, eidx)


def _sc_triples(xout, ea, eidx):
    mesh = plsc.VectorSubcoreMesh(core_axis_name="c", subcore_axis_name="s")

    @functools.partial(
        pl.kernel,
        out_type=jax.ShapeDtypeStruct((E, 3 * D), jnp.float32),
        mesh=mesh,
        scratch_types=[
            pltpu.VMEM((EPW,), jnp.int32),
            pltpu.VMEM((EPW,), jnp.int32),
            [pltpu.VMEM((C, 3 * D), jnp.float32) for _ in range(2)],
            [pltpu.VMEM((C,), jnp.int32) for _ in range(2)],
            [pltpu.VMEM((C,), jnp.int32) for _ in range(2)],
            [pltpu.VMEM((C,), jnp.int32) for _ in range(2)],
            [pltpu.SemaphoreType.DMA for _ in range(2)],
            [pltpu.SemaphoreType.DMA for _ in range(2)],
            [pltpu.SemaphoreType.DMA for _ in range(2)],
            [pltpu.SemaphoreType.DMA for _ in range(2)],
        ],
    )
    def k(xout_hbm, ea_hbm, eidx_hbm, out_hbm,
          psrc, pdst, ivl, gis, gid, oxe, gss, gse, gsd, ssc):
        cid = lax.axis_index("c")
        sid = lax.axis_index("s")
        wid = sid * NC + cid
        eb = wid * EPW
        pltpu.sync_copy(eidx_hbm.at[pl.ds(eb, EPW)], psrc)
        pltpu.sync_copy(eidx_hbm.at[pl.ds(E + eb, EPW)], pdst)

        def fill_idx(bb, cb):
            for j in range(C // L):
                lane = lax.iota(jnp.int32, L)
                gis[bb][pl.ds(j * L, L)] = psrc[pl.ds(cb + j * L, L)]
                gid[bb][pl.ds(j * L, L)] = pdst[pl.ds(cb + j * L, L)]
                oxe[bb][pl.ds(j * L, L)] = eb + cb + j * L + lane

        def issue_gathers(bb):
            pltpu.async_copy(xout_hbm.at[gis[bb]],
                             ivl[bb].at[:, pl.ds(0, D)], gss[bb])
            pltpu.async_copy(ea_hbm.at[oxe[bb]],
                             ivl[bb].at[:, pl.ds(D, D)], gse[bb])
            pltpu.async_copy(xout_hbm.at[gid[bb]],
                             ivl[bb].at[:, pl.ds(2 * D, D)], gsd[bb])

        def wait_gathers(bb):
            pltpu.make_async_copy(xout_hbm.at[gis[bb]],
                                  ivl[bb].at[:, pl.ds(0, D)], gss[bb]).wait()
            pltpu.make_async_copy(ea_hbm.at[oxe[bb]],
                                  ivl[bb].at[:, pl.ds(D, D)], gse[bb]).wait()
            pltpu.make_async_copy(xout_hbm.at[gid[bb]],
                                  ivl[bb].at[:, pl.ds(2 * D, D)],
                                  gsd[bb]).wait()

        fill_idx(0, 0)
        issue_gathers(0)

        def body(k2, _):
            for b in range(2):
                c = k2 * 2 + b
                ob = 1 - b
                wait_gathers(b)
                pltpu.async_copy(ivl[b], out_hbm.at[oxe[b]], ssc[b])

                @pl.when(c + 1 < NCH)
                def _pf():
                    @pl.when(c >= 1)
                    def _drain():
                        pltpu.make_async_copy(
                            ivl[ob], out_hbm.at[oxe[ob]], ssc[ob]).wait()
                    cb1 = lax.min((c + 1) * C, CLAMP)
                    fill_idx(ob, cb1)
                    issue_gathers(ob)
            return 0

        lax.fori_loop(0, NCH // 2, body, 0)
        pltpu.make_async_copy(ivl[0], out_hbm.at[oxe[0]], ssc[0]).wait()
        pltpu.make_async_copy(ivl[1], out_hbm.at[oxe[1]], ssc[1]).wait()

    return k(xout, ea, eidx)


def kernel(concept_embedding, relation_embedding, edge_index, edge_relation,
           edge_weight, W_msg, W_self, W_upd):
    x = concept_embedding
    eidx = edge_index.reshape(2 * E)
    rel2 = edge_relation.reshape(NEB, 1, EBM)
    w2 = edge_weight.reshape(NEB, 1, EBM)
    xr = jnp.concatenate(
        [x, relation_embedding,
         jnp.zeros((10240 - N - R, D), jnp.float32)], axis=0)
    xrm = _tc_matmul(xr, W_msg, bm=2048)
    em = lax.slice(xrm, (N, 0), (N + R, D))
    eam = _tc_edge_attr(rel2, w2, em)
    ea = _tc_edge_attr(rel2, w2, relation_embedding)
    aggp = _sc_agg(xrm, eam, eidx)
    xout = _tc_update(aggp, x, W_upd, W_self)
    return _sc_triples(xout, ea, eidx)


# speedup vs baseline: 6.2083x; 1.0826x over previous
import functools

import jax
import jax.numpy as jnp
from jax import lax
from jax.experimental import pallas as pl
from jax.experimental.pallas import tpu as pltpu
from jax.experimental.pallas import tpu_sc as plsc

N = 10000
E = 160000
D = 128
R = 40

NC, NS, L = 2, 16, 16
NW = NC * NS
EPW = E // NW
C = 96
NCH = 54
CLAMP = EPW - C
G = D // L
ZB = 632
NPAD = NS * ZB

EBM = 6400
NEB = E // EBM


def _mm_body(a_ref, w_ref, o_ref):
    o_ref[...] = jnp.dot(a_ref[...], w_ref[...],
                         preferred_element_type=jnp.float32)


def _tc_matmul(a, w, bm):
    m = a.shape[0]
    return pl.pallas_call(
        _mm_body,
        grid=(m // bm,),
        in_specs=[pl.BlockSpec((bm, D), lambda i: (i, 0)),
                  pl.BlockSpec((D, D), lambda i: (0, 0))],
        out_specs=pl.BlockSpec((bm, D), lambda i: (i, 0)),
        out_shape=jax.ShapeDtypeStruct((m, D), jnp.float32),
    )(a, w)


def _attr_body(rel_ref, w_ref, tab_ref, o_ref):
    relb = rel_ref[0, 0]
    wb = w_ref[0, 0]
    onehot = (relb[:, None]
              == lax.broadcasted_iota(jnp.int32, (EBM, R), 1))
    o_ref[...] = wb[:, None] * jnp.dot(
        onehot.astype(jnp.float32), tab_ref[...],
        preferred_element_type=jnp.float32)


def _tc_edge_attr(rel2, w2, tab):
    return pl.pallas_call(
        _attr_body,
        grid=(NEB,),
        in_specs=[pl.BlockSpec((1, 1, EBM), lambda i: (i, 0, 0)),
                  pl.BlockSpec((1, 1, EBM), lambda i: (i, 0, 0)),
                  pl.BlockSpec((R, D), lambda i: (0, 0))],
        out_specs=pl.BlockSpec((EBM, D), lambda i: (i, 0)),
        out_shape=jax.ShapeDtypeStruct((E, D), jnp.float32),
    )(rel2, w2, tab)


def _upd_body(agg_ref, x_ref, wu_ref, ws_ref, o_ref):
    a = agg_ref[0] + agg_ref[1]
    o_ref[...] = jnp.maximum(
        jnp.dot(a, wu_ref[...], preferred_element_type=jnp.float32)
        + jnp.dot(x_ref[...], ws_ref[...], preferred_element_type=jnp.float32),
        0.0)


def _tc_update(aggp, x, w_upd, w_self, bm=400):
    return pl.pallas_call(
        _upd_body,
        grid=(N // bm,),
        in_specs=[pl.BlockSpec((NC, bm, D), lambda i: (0, i, 0)),
                  pl.BlockSpec((bm, D), lambda i: (i, 0)),
                  pl.BlockSpec((D, D), lambda i: (0, 0)),
                  pl.BlockSpec((D, D), lambda i: (0, 0))],
        out_specs=pl.BlockSpec((bm, D), lambda i: (i, 0)),
        out_shape=jax.ShapeDtypeStruct((N, D), jnp.float32),
    )(aggp, x, w_upd, w_self)


def _sc_agg(xm, eam, eidx):
    mesh = plsc.VectorSubcoreMesh(core_axis_name="c", subcore_axis_name="s")

    @functools.partial(
        pl.kernel,
        out_type=jax.ShapeDtypeStruct((NC, N, D), jnp.float32),
        mesh=mesh,
        scratch_types=[
            pltpu.VMEM((EPW,), jnp.int32),
            pltpu.VMEM((EPW,), jnp.int32),
            [pltpu.VMEM((C, D), jnp.float32) for _ in range(3)],
            [pltpu.VMEM((C,), jnp.int32) for _ in range(3)],
            [pltpu.VMEM((C,), jnp.int32) for _ in range(3)],
            [pltpu.VMEM((C,), jnp.int32) for _ in range(3)],
            [pltpu.SemaphoreType.DMA for _ in range(3)],
            [pltpu.SemaphoreType.DMA for _ in range(3)],
            [pltpu.SemaphoreType.DMA for _ in range(3)],
            pltpu.VMEM_SHARED((NPAD, D), jnp.float32),
        ],
    )
    def k(xm_hbm, eam_hbm, eidx_hbm, out_hbm,
          psrc, pdst, bufs, gidx, lidx, sdix, gsem, asem, ssem, agg_sh):
        buf = bufs[0]
        cid = lax.axis_index("c")
        sid = lax.axis_index("s")
        wid = sid * NC + cid

        def zrow(i, _):
            r = i // G
            g = i % G
            buf[r, pl.ds(g * L, L)] = jnp.zeros((L,), jnp.float32)
            return 0
        lax.fori_loop(0, C * G, zrow, 0)
        base_z = sid * ZB
        for b in range(ZB // C):
            pltpu.sync_copy(buf, agg_sh.at[pl.ds(base_z + b * C, C)])
        pltpu.sync_copy(buf.at[pl.ds(0, ZB % C)],
                        agg_sh.at[pl.ds(base_z + (ZB // C) * C, ZB % C)])
        plsc.subcore_barrier()

        eb = wid * EPW
        pltpu.sync_copy(eidx_hbm.at[pl.ds(eb, EPW)], psrc)
        pltpu.sync_copy(eidx_hbm.at[pl.ds(E + eb, EPW)], pdst)

        def fill_idx(bb, cb):
            for j in range(C // L):
                lane = lax.iota(jnp.int32, L)
                gidx[bb][pl.ds(j * L, L)] = psrc[pl.ds(cb + j * L, L)]
                lidx[bb][pl.ds(j * L, L)] = eb + cb + j * L + lane

        def zero_buf(bb):
            def zb(r, _):
                for g in range(G):
                    bufs[bb][r, pl.ds(g * L, L)] = jnp.zeros((L,), jnp.float32)
                return 0
            lax.fori_loop(0, C, zb, 0)

        def issue_adds(bb):
            pltpu.async_copy(xm_hbm.at[gidx[bb]], bufs[bb], gsem[bb],
                             add=True)
            pltpu.async_copy(eam_hbm.at[lidx[bb]], bufs[bb], asem[bb],
                             add=True)

        def wait_adds(bb):
            pltpu.make_async_copy(
                xm_hbm.at[gidx[bb]], bufs[bb], gsem[bb]).wait()
            pltpu.make_async_copy(
                eam_hbm.at[lidx[bb]], bufs[bb], asem[bb]).wait()

        fill_idx(0, 0)
        issue_adds(0)
        zero_buf(1)
        fill_idx(1, C)
        issue_adds(1)

        def body(k3, _):
            for b in range(3):
                c = k3 * 3 + b
                cb = lax.min(c * C, CLAMP)
                nb = (b + 2) % 3
                wait_adds(b)

                thr = lax.min(c * C, EPW) - cb
                for j in range(C // L):
                    v = pdst[pl.ds(cb + j * L, L)]
                    lanes = j * L + lax.iota(jnp.int32, L)
                    sdix[b][pl.ds(j * L, L)] = jnp.where(
                        lanes >= thr, v, jnp.full((L,), N, jnp.int32))

                def sub(r, _):
                    for g in range(G):
                        v = bufs[b][r, pl.ds(g * L, L)]
                        bufs[b][r, pl.ds(g * L, L)] = jnp.maximum(v, 0.0)
                    return 0
                lax.fori_loop(0, C, sub, 0)

                pltpu.async_copy(bufs[b], agg_sh.at[sdix[b]], ssem[b],
                                 add=True)

                @pl.when(c + 2 < NCH)
                def _pf():
                    @pl.when(c >= 1)
                    def _drain():
                        pltpu.make_async_copy(
                            bufs[nb], agg_sh.at[sdix[nb]], ssem[nb]).wait()
                    zero_buf(nb)
                    cb1 = lax.min((c + 2) * C, CLAMP)
                    fill_idx(nb, cb1)
                    issue_adds(nb)
            return 0

        lax.fori_loop(0, NCH // 3, body, 0)
        for b in range(3):
            pltpu.make_async_copy(bufs[b], agg_sh.at[sdix[b]], ssem[b]).wait()
        plsc.subcore_barrier()

        base_o = sid * ZB

        @pl.when(sid < NS - 1)
        def _w_full():
            pltpu.sync_copy(agg_sh.at[pl.ds(base_o, ZB)],
                            out_hbm.at[cid, pl.ds(base_o, ZB)])

        @pl.when(sid == NS - 1)
        def _w_last():
            rows = N - (NS - 1) * ZB
            pltpu.sync_copy(agg_sh.at[pl.ds(base_o, rows)],
                            out_hbm.at[cid, pl.ds(base_o, rows)])

    return k(xm, eam, eidx)


def _sc_triples(xout, ea, eidx):
    mesh = plsc.VectorSubcoreMesh(core_axis_name="c", subcore_axis_name="s")

    @functools.partial(
        pl.kernel,
        out_type=jax.ShapeDtypeStruct((E, 3 * D), jnp.float32),
        mesh=mesh,
        scratch_types=[
            pltpu.VMEM((EPW,), jnp.int32),
            pltpu.VMEM((EPW,), jnp.int32),
            [pltpu.VMEM((C, 3 * D), jnp.float32) for _ in range(3)],
            [pltpu.VMEM((C,), jnp.int32) for _ in range(3)],
            [pltpu.VMEM((C,), jnp.int32) for _ in range(3)],
            [pltpu.VMEM((C,), jnp.int32) for _ in range(3)],
            [pltpu.SemaphoreType.DMA for _ in range(3)],
            [pltpu.SemaphoreType.DMA for _ in range(3)],
            [pltpu.SemaphoreType.DMA for _ in range(3)],
            [pltpu.SemaphoreType.DMA for _ in range(3)],
        ],
    )
    def k(xout_hbm, ea_hbm, eidx_hbm, out_hbm,
          psrc, pdst, ivl, gis, gid, oxe, gss, gse, gsd, ssc):
        cid = lax.axis_index("c")
        sid = lax.axis_index("s")
        wid = sid * NC + cid
        eb = wid * EPW
        pltpu.sync_copy(eidx_hbm.at[pl.ds(eb, EPW)], psrc)
        pltpu.sync_copy(eidx_hbm.at[pl.ds(E + eb, EPW)], pdst)

        def fill_idx(bb, cb):
            for j in range(C // L):
                lane = lax.iota(jnp.int32, L)
                gis[bb][pl.ds(j * L, L)] = psrc[pl.ds(cb + j * L, L)]
                gid[bb][pl.ds(j * L, L)] = pdst[pl.ds(cb + j * L, L)]
                oxe[bb][pl.ds(j * L, L)] = eb + cb + j * L + lane

        def issue_gathers(bb):
            pltpu.async_copy(xout_hbm.at[gis[bb]],
                             ivl[bb].at[:, pl.ds(0, D)], gss[bb])
            pltpu.async_copy(ea_hbm.at[oxe[bb]],
                             ivl[bb].at[:, pl.ds(D, D)], gse[bb])
            pltpu.async_copy(xout_hbm.at[gid[bb]],
                             ivl[bb].at[:, pl.ds(2 * D, D)], gsd[bb])

        def wait_gathers(bb):
            pltpu.make_async_copy(xout_hbm.at[gis[bb]],
                                  ivl[bb].at[:, pl.ds(0, D)], gss[bb]).wait()
            pltpu.make_async_copy(ea_hbm.at[oxe[bb]],
                                  ivl[bb].at[:, pl.ds(D, D)], gse[bb]).wait()
            pltpu.make_async_copy(xout_hbm.at[gid[bb]],
                                  ivl[bb].at[:, pl.ds(2 * D, D)],
                                  gsd[bb]).wait()

        fill_idx(0, 0)
        issue_gathers(0)
        fill_idx(1, C)
        issue_gathers(1)

        def body(k3, _):
            for b in range(3):
                c = k3 * 3 + b
                nb = (b + 2) % 3
                wait_gathers(b)
                pltpu.async_copy(ivl[b], out_hbm.at[oxe[b]], ssc[b])

                @pl.when(c + 2 < NCH)
                def _pf():
                    @pl.when(c >= 1)
                    def _drain():
                        pltpu.make_async_copy(
                            ivl[nb], out_hbm.at[oxe[nb]], ssc[nb]).wait()
                    cb1 = lax.min((c + 2) * C, CLAMP)
                    fill_idx(nb, cb1)
                    issue_gathers(nb)
            return 0

        lax.fori_loop(0, NCH // 3, body, 0)
        for b in range(3):
            pltpu.make_async_copy(ivl[b], out_hbm.at[oxe[b]], ssc[b]).wait()

    return k(xout, ea, eidx)


def kernel(concept_embedding, relation_embedding, edge_index, edge_relation,
           edge_weight, W_msg, W_self, W_upd):
    x = concept_embedding
    eidx = edge_index.reshape(2 * E)
    rel2 = edge_relation.reshape(NEB, 1, EBM)
    w2 = edge_weight.reshape(NEB, 1, EBM)
    xr = jnp.concatenate(
        [x, relation_embedding,
         jnp.zeros((10240 - N - R, D), jnp.float32)], axis=0)
    xrm = _tc_matmul(xr, W_msg, bm=2048)
    em = lax.slice(xrm, (N, 0), (N + R, D))
    eam = _tc_edge_attr(rel2, w2, em)
    ea = _tc_edge_attr(rel2, w2, relation_embedding)
    aggp = _sc_agg(xrm, eam, eidx)
    xout = _tc_update(aggp, x, W_upd, W_self)
    return _sc_triples(xout, ea, eidx)


# speedup vs baseline: 6.3737x; 1.0266x over previous
import functools

import jax
import jax.numpy as jnp
from jax import lax
from jax.experimental import pallas as pl
from jax.experimental.pallas import tpu as pltpu
from jax.experimental.pallas import tpu_sc as plsc

N = 10000
E = 160000
D = 128
R = 40

NC, NS, L = 2, 16, 16
NW = NC * NS
EPW = E // NW
C = 96
NCH = 54
CLAMP = EPW - C
G = D // L
ZB = 632
NPAD = NS * ZB

EBM = 8000
NEB = E // EBM


def _mm_body(a_ref, w_ref, o_ref):
    o_ref[...] = jnp.dot(a_ref[...], w_ref[...],
                         preferred_element_type=jnp.float32)


def _tc_matmul(a, w, bm):
    m = a.shape[0]
    return pl.pallas_call(
        _mm_body,
        grid=(m // bm,),
        in_specs=[pl.BlockSpec((bm, D), lambda i: (i, 0)),
                  pl.BlockSpec((D, D), lambda i: (0, 0))],
        out_specs=pl.BlockSpec((bm, D), lambda i: (i, 0)),
        out_shape=jax.ShapeDtypeStruct((m, D), jnp.float32),
    )(a, w)


def _attr_body(rel_ref, w_ref, tab_ref, o_ref):
    relb = rel_ref[0, 0]
    wb = w_ref[0, 0]
    onehot = (relb[:, None]
              == lax.broadcasted_iota(jnp.int32, (EBM, R), 1))
    o_ref[...] = wb[:, None] * jnp.dot(
        onehot.astype(jnp.float32), tab_ref[...],
        preferred_element_type=jnp.float32)


def _tc_edge_attr(rel2, w2, tab):
    return pl.pallas_call(
        _attr_body,
        grid=(NEB,),
        in_specs=[pl.BlockSpec((1, 1, EBM), lambda i: (i, 0, 0)),
                  pl.BlockSpec((1, 1, EBM), lambda i: (i, 0, 0)),
                  pl.BlockSpec((R, D), lambda i: (0, 0))],
        out_specs=pl.BlockSpec((EBM, D), lambda i: (i, 0)),
        out_shape=jax.ShapeDtypeStruct((E, D), jnp.float32),
    )(rel2, w2, tab)


def _upd_body(agg_ref, x_ref, wu_ref, ws_ref, o_ref):
    a = agg_ref[0] + agg_ref[1]
    o_ref[...] = jnp.maximum(
        jnp.dot(a, wu_ref[...], preferred_element_type=jnp.float32)
        + jnp.dot(x_ref[...], ws_ref[...], preferred_element_type=jnp.float32),
        0.0)


def _tc_update(aggp, x, w_upd, w_self, bm=2000):
    return pl.pallas_call(
        _upd_body,
        grid=(N // bm,),
        in_specs=[pl.BlockSpec((NC, bm, D), lambda i: (0, i, 0)),
                  pl.BlockSpec((bm, D), lambda i: (i, 0)),
                  pl.BlockSpec((D, D), lambda i: (0, 0)),
                  pl.BlockSpec((D, D), lambda i: (0, 0))],
        out_specs=pl.BlockSpec((bm, D), lambda i: (i, 0)),
        out_shape=jax.ShapeDtypeStruct((N, D), jnp.float32),
    )(aggp, x, w_upd, w_self)


def _sc_agg(xm, eam, eidx):
    mesh = plsc.VectorSubcoreMesh(core_axis_name="c", subcore_axis_name="s")

    @functools.partial(
        pl.kernel,
        out_type=jax.ShapeDtypeStruct((NC, N, D), jnp.float32),
        mesh=mesh,
        scratch_types=[
            pltpu.VMEM((EPW,), jnp.int32),
            pltpu.VMEM((EPW,), jnp.int32),
            [pltpu.VMEM((C, D), jnp.float32) for _ in range(3)],
            [pltpu.VMEM((C,), jnp.int32) for _ in range(3)],
            [pltpu.VMEM((C,), jnp.int32) for _ in range(3)],
            [pltpu.VMEM((C,), jnp.int32) for _ in range(3)],
            [pltpu.SemaphoreType.DMA for _ in range(3)],
            [pltpu.SemaphoreType.DMA for _ in range(3)],
            [pltpu.SemaphoreType.DMA for _ in range(3)],
            pltpu.VMEM_SHARED((NPAD, D), jnp.float32),
        ],
    )
    def k(xm_hbm, eam_hbm, eidx_hbm, out_hbm,
          psrc, pdst, bufs, gidx, lidx, sdix, gsem, asem, ssem, agg_sh):
        buf = bufs[0]
        cid = lax.axis_index("c")
        sid = lax.axis_index("s")
        wid = sid * NC + cid

        def zrow(i, _):
            r = i // G
            g = i % G
            buf[r, pl.ds(g * L, L)] = jnp.zeros((L,), jnp.float32)
            return 0
        lax.fori_loop(0, C * G, zrow, 0)
        base_z = sid * ZB
        for b in range(ZB // C):
            pltpu.sync_copy(buf, agg_sh.at[pl.ds(base_z + b * C, C)])
        pltpu.sync_copy(buf.at[pl.ds(0, ZB % C)],
                        agg_sh.at[pl.ds(base_z + (ZB // C) * C, ZB % C)])
        plsc.subcore_barrier()

        eb = wid * EPW
        pltpu.sync_copy(eidx_hbm.at[pl.ds(eb, EPW)], psrc)
        pltpu.sync_copy(eidx_hbm.at[pl.ds(E + eb, EPW)], pdst)

        def fill_idx(bb, cb):
            for j in range(C // L):
                lane = lax.iota(jnp.int32, L)
                gidx[bb][pl.ds(j * L, L)] = psrc[pl.ds(cb + j * L, L)]
                lidx[bb][pl.ds(j * L, L)] = eb + cb + j * L + lane

        def zero_buf(bb):
            def zb(r, _):
                for g in range(G):
                    bufs[bb][r, pl.ds(g * L, L)] = jnp.zeros((L,), jnp.float32)
                return 0
            lax.fori_loop(0, C, zb, 0)

        def issue_adds(bb):
            pltpu.async_copy(xm_hbm.at[gidx[bb]], bufs[bb], gsem[bb],
                             add=True)
            pltpu.async_copy(eam_hbm.at[lidx[bb]], bufs[bb], asem[bb],
                             add=True)

        def wait_adds(bb):
            pltpu.make_async_copy(
                xm_hbm.at[gidx[bb]], bufs[bb], gsem[bb]).wait()
            pltpu.make_async_copy(
                eam_hbm.at[lidx[bb]], bufs[bb], asem[bb]).wait()

        fill_idx(0, 0)
        issue_adds(0)
        zero_buf(1)
        fill_idx(1, C)
        issue_adds(1)

        def body(k3, _):
            for b in range(3):
                c = k3 * 3 + b
                cb = lax.min(c * C, CLAMP)
                nb = (b + 2) % 3
                wait_adds(b)

                thr = lax.min(c * C, EPW) - cb
                for j in range(C // L):
                    v = pdst[pl.ds(cb + j * L, L)]
                    lanes = j * L + lax.iota(jnp.int32, L)
                    sdix[b][pl.ds(j * L, L)] = jnp.where(
                        lanes >= thr, v, jnp.full((L,), N, jnp.int32))

                def sub(r, _):
                    for g in range(G):
                        v = bufs[b][r, pl.ds(g * L, L)]
                        bufs[b][r, pl.ds(g * L, L)] = jnp.maximum(v, 0.0)
                    return 0
                lax.fori_loop(0, C, sub, 0)

                pltpu.async_copy(bufs[b], agg_sh.at[sdix[b]], ssem[b],
                                 add=True)

                @pl.when(c + 2 < NCH)
                def _pf():
                    @pl.when(c >= 1)
                    def _drain():
                        pltpu.make_async_copy(
                            bufs[nb], agg_sh.at[sdix[nb]], ssem[nb]).wait()
                    zero_buf(nb)
                    cb1 = lax.min((c + 2) * C, CLAMP)
                    fill_idx(nb, cb1)
                    issue_adds(nb)
            return 0

        lax.fori_loop(0, NCH // 3, body, 0)
        for b in range(3):
            pltpu.make_async_copy(bufs[b], agg_sh.at[sdix[b]], ssem[b]).wait()
        plsc.subcore_barrier()

        base_o = sid * ZB

        @pl.when(sid < NS - 1)
        def _w_full():
            pltpu.sync_copy(agg_sh.at[pl.ds(base_o, ZB)],
                            out_hbm.at[cid, pl.ds(base_o, ZB)])

        @pl.when(sid == NS - 1)
        def _w_last():
            rows = N - (NS - 1) * ZB
            pltpu.sync_copy(agg_sh.at[pl.ds(base_o, rows)],
                            out_hbm.at[cid, pl.ds(base_o, rows)])

    return k(xm, eam, eidx)


def _sc_triples(xout, ea, eidx):
    mesh = plsc.VectorSubcoreMesh(core_axis_name="c", subcore_axis_name="s")

    @functools.partial(
        pl.kernel,
        out_type=jax.ShapeDtypeStruct((E, 3 * D), jnp.float32),
        mesh=mesh,
        scratch_types=[
            pltpu.VMEM((EPW,), jnp.int32),
            pltpu.VMEM((EPW,), jnp.int32),
            [pltpu.VMEM((C, 3 * D), jnp.float32) for _ in range(3)],
            [pltpu.VMEM((C,), jnp.int32) for _ in range(3)],
            [pltpu.VMEM((C,), jnp.int32) for _ in range(3)],
            [pltpu.VMEM((C,), jnp.int32) for _ in range(3)],
            [pltpu.SemaphoreType.DMA for _ in range(3)],
            [pltpu.SemaphoreType.DMA for _ in range(3)],
            [pltpu.SemaphoreType.DMA for _ in range(3)],
            [pltpu.SemaphoreType.DMA for _ in range(3)],
        ],
    )
    def k(xout_hbm, ea_hbm, eidx_hbm, out_hbm,
          psrc, pdst, ivl, gis, gid, oxe, gss, gse, gsd, ssc):
        cid = lax.axis_index("c")
        sid = lax.axis_index("s")
        wid = sid * NC + cid
        eb = wid * EPW
        pltpu.sync_copy(eidx_hbm.at[pl.ds(eb, EPW)], psrc)
        pltpu.sync_copy(eidx_hbm.at[pl.ds(E + eb, EPW)], pdst)

        def fill_idx(bb, cb):
            for j in range(C // L):
                lane = lax.iota(jnp.int32, L)
                gis[bb][pl.ds(j * L, L)] = psrc[pl.ds(cb + j * L, L)]
                gid[bb][pl.ds(j * L, L)] = pdst[pl.ds(cb + j * L, L)]
                oxe[bb][pl.ds(j * L, L)] = eb + cb + j * L + lane

        def issue_gathers(bb):
            pltpu.async_copy(xout_hbm.at[gis[bb]],
                             ivl[bb].at[:, pl.ds(0, D)], gss[bb])
            pltpu.async_copy(ea_hbm.at[oxe[bb]],
                             ivl[bb].at[:, pl.ds(D, D)], gse[bb])
            pltpu.async_copy(xout_hbm.at[gid[bb]],
                             ivl[bb].at[:, pl.ds(2 * D, D)], gsd[bb])

        def wait_gathers(bb):
            pltpu.make_async_copy(xout_hbm.at[gis[bb]],
                                  ivl[bb].at[:, pl.ds(0, D)], gss[bb]).wait()
            pltpu.make_async_copy(ea_hbm.at[oxe[bb]],
                                  ivl[bb].at[:, pl.ds(D, D)], gse[bb]).wait()
            pltpu.make_async_copy(xout_hbm.at[gid[bb]],
                                  ivl[bb].at[:, pl.ds(2 * D, D)],
                                  gsd[bb]).wait()

        fill_idx(0, 0)
        issue_gathers(0)
        fill_idx(1, C)
        issue_gathers(1)

        def body(k3, _):
            for b in range(3):
                c = k3 * 3 + b
                nb = (b + 2) % 3
                wait_gathers(b)
                pltpu.async_copy(ivl[b], out_hbm.at[oxe[b]], ssc[b])

                @pl.when(c + 2 < NCH)
                def _pf():
                    @pl.when(c >= 1)
                    def _drain():
                        pltpu.make_async_copy(
                            ivl[nb], out_hbm.at[oxe[nb]], ssc[nb]).wait()
                    cb1 = lax.min((c + 2) * C, CLAMP)
                    fill_idx(nb, cb1)
                    issue_gathers(nb)
            return 0

        lax.fori_loop(0, NCH // 3, body, 0)
        for b in range(3):
            pltpu.make_async_copy(ivl[b], out_hbm.at[oxe[b]], ssc[b]).wait()

    return k(xout, ea, eidx)


def kernel(concept_embedding, relation_embedding, edge_index, edge_relation,
           edge_weight, W_msg, W_self, W_upd):
    x = concept_embedding
    eidx = edge_index.reshape(2 * E)
    rel2 = edge_relation.reshape(NEB, 1, EBM)
    w2 = edge_weight.reshape(NEB, 1, EBM)
    xr = jnp.concatenate(
        [x, relation_embedding,
         jnp.zeros((10240 - N - R, D), jnp.float32)], axis=0)
    xrm = _tc_matmul(xr, W_msg, bm=2048)
    em = lax.slice(xrm, (N, 0), (N + R, D))
    eam = _tc_edge_attr(rel2, w2, em)
    ea = _tc_edge_attr(rel2, w2, relation_embedding)
    aggp = _sc_agg(xrm, eam, eidx)
    xout = _tc_update(aggp, x, W_upd, W_self)
    return _sc_triples(xout, ea, eidx)
